# baseline (speedup 1.0000x reference)
import functools

import jax
import jax.numpy as jnp
from jax import lax
from jax.experimental import pallas as pl
from jax.experimental.pallas import tpu as pltpu
from jax.experimental.pallas import tpu_sc as plsc

N = 10000
E = 320000
D = 128

NC = 2
NS = 16
NW = NC * NS
EPT = E // NW
RPT = N // NS

C = 128
NCH = 80
EPT_P = NCH * C
E_P = NW * EPT_P
NBUF = 8
NROW = N + 400
RPT_Z = NROW // NS

def _deg_body(dst_hbm, out_hbm, dst_v, ones_v, zero_v, acc_sh):
    c = lax.axis_index("c")
    s = lax.axis_index("s")
    pltpu.sync_copy(dst_hbm.at[c, s], dst_v)

    def fill_ones(i, _):
        ones_v[i, :] = jnp.full((16,), 1.0, jnp.float32)
        return 0

    lax.fori_loop(0, C, fill_ones, 0)

    def fill_zero(i, _):
        zero_v[i, :] = jnp.zeros((16,), jnp.float32)
        return 0

    lax.fori_loop(0, 25, fill_zero, 0)

    def zcp(k, _):
        pltpu.sync_copy(zero_v, acc_sh.at[pl.ds(s * RPT_Z + k * 25, 25)])
        return 0

    lax.fori_loop(0, RPT_Z // 25, zcp, 0)
    plsc.subcore_barrier()

    def body(j, _):
        pltpu.sync_copy(ones_v, acc_sh.at[dst_v.at[j]], add=True)
        return 0

    lax.fori_loop(0, NCH, body, 0)
    plsc.subcore_barrier()
    pltpu.sync_copy(acc_sh.at[pl.ds(s * RPT, RPT)], out_hbm.at[c, s])


@functools.cache
def _deg_call():
    return pl.kernel(
        _deg_body,
        out_type=jax.ShapeDtypeStruct((NC, NS, RPT, 16), jnp.float32),
        mesh=plsc.VectorSubcoreMesh(core_axis_name="c", subcore_axis_name="s"),
        scratch_types=[
            pltpu.VMEM((NCH, C), jnp.int32),
            pltpu.VMEM((C, 16), jnp.float32),
            pltpu.VMEM((25, 16), jnp.float32),
            pltpu.VMEM_SHARED((NROW, 16), jnp.float32),
        ],
        compiler_params=pltpu.CompilerParams(use_tc_tiling_on_sc=False),
    )


DH = D // 2


def _layer_body(h_hbm, src_hbm, dst_hbm, out_hbm, src_v, dst_v, rows_v, zero_v,
                acc_sh, sem0, sem1):
    c = lax.axis_index("c")
    s = lax.axis_index("s")
    pltpu.sync_copy(src_hbm.at[c, s], src_v)
    pltpu.sync_copy(dst_hbm.at[c, s], dst_v)

    def fz(i, _):
        for jj in range(DH // 16):
            zero_v[i, pl.ds(jj * 16, 16)] = jnp.zeros((16,), jnp.float32)
        return 0

    lax.fori_loop(0, 25, fz, 0)

    for half in range(2):
        def zcp(k, _):
            pltpu.sync_copy(zero_v, acc_sh.at[pl.ds(s * RPT_Z + k * 25, 25)])
            return 0

        lax.fori_loop(0, RPT_Z // 25, zcp, 0)
        plsc.subcore_barrier()

        h_half = h_hbm.at[half]
        for b in range(NBUF):
            pltpu.async_copy(h_half.at[src_v.at[b]], rows_v.at[b], sem0)

        def body(g, _):
            j0 = NBUF * g
            for b in range(NBUF):
                pltpu.make_async_copy(h_half.at[src_v.at[j0 + b]],
                                      rows_v.at[b], sem0).wait()
                pltpu.async_copy(rows_v.at[b], acc_sh.at[dst_v.at[j0 + b]],
                                 sem1, add=True)
            for b in range(NBUF):
                nj = j0 + NBUF + b

                @pl.when(nj < NCH)
                def _():
                    pltpu.make_async_copy(rows_v.at[b],
                                          acc_sh.at[dst_v.at[j0 + b]],
                                          sem1).wait()
                    pltpu.async_copy(h_half.at[src_v.at[nj]], rows_v.at[b], sem0)

            return 0

        lax.fori_loop(0, NCH // NBUF, body, 0)
        for b in range(NBUF):
            pltpu.make_async_copy(rows_v.at[b], acc_sh.at[dst_v.at[b]],
                                  sem1).wait()
        plsc.subcore_barrier()
        pltpu.sync_copy(
            acc_sh.at[pl.ds(s * RPT, RPT)],
            out_hbm.at[c, pl.ds(s * RPT, RPT), pl.ds(half * DH, DH)])


@functools.cache
def _layer_call():
    return pl.kernel(
        _layer_body,
        out_type=jax.ShapeDtypeStruct((NC, N, D), jnp.float32),
        mesh=plsc.VectorSubcoreMesh(core_axis_name="c", subcore_axis_name="s"),
        scratch_types=[
            pltpu.VMEM((NCH, C), jnp.int32),
            pltpu.VMEM((NCH, C), jnp.int32),
            pltpu.VMEM((NBUF, C, DH), jnp.float32),
            pltpu.VMEM((25, DH), jnp.float32),
            pltpu.VMEM_SHARED((NROW, DH), jnp.float32),
            pltpu.SemaphoreType.DMA,
            pltpu.SemaphoreType.DMA,
        ],
        compiler_params=pltpu.CompilerParams(use_tc_tiling_on_sc=False),
    )


R = 1000
GRID = N // R


def _dinv_of(dp_ref):
    deg = dp_ref[0, :, 0:1] + dp_ref[1, :, 0:1] + 1.0
    return lax.rsqrt(deg)


def _write_halves(o_ref, res):
    o_ref[0] = res[:, :DH]
    o_ref[1] = res[:, DH:]


def _pre_halves(dinv_ref, a_ref, h_ref, b_ref):
    asum = a_ref[0] + a_ref[1]
    return [
        (asum[:, k * DH:(k + 1) * DH] + h_ref[k])
        * dinv_ref[:, k * DH:(k + 1) * DH]
        + b_ref[0, k * DH:(k + 1) * DH][None, :]
        for k in range(2)
    ]


def _t1_body(dp_ref, x_ref, w_ref, o_ref, dinv_ref):
    dinv = _dinv_of(dp_ref)
    res = jnp.dot(x_ref[...], w_ref[...],
                  preferred_element_type=jnp.float32) * dinv
    _write_halves(o_ref, res)
    dinv_ref[...] = jnp.broadcast_to(dinv, (R, D))


_t1_call = pl.pallas_call(
    _t1_body,
    grid=(GRID,),
    in_specs=[
        pl.BlockSpec((NC, R, 16), lambda i: (0, i, 0)),
        pl.BlockSpec((R, D), lambda i: (i, 0)),
        pl.BlockSpec((D, D), lambda i: (0, 0)),
    ],
    out_specs=[
        pl.BlockSpec((2, R, DH), lambda i: (0, i, 0)),
        pl.BlockSpec((R, D), lambda i: (i, 0)),
    ],
    out_shape=[
        jax.ShapeDtypeStruct((2, N, DH), jnp.float32),
        jax.ShapeDtypeStruct((N, D), jnp.float32),
    ],
)


def _t2_body(dinv_ref, a_ref, h_ref, b_ref, w_ref, o_ref):
    pre = _pre_halves(dinv_ref, a_ref, h_ref, b_ref)
    res = (jnp.dot(jnp.maximum(pre[0], 0.0), w_ref[:DH, :],
                   preferred_element_type=jnp.float32) +
           jnp.dot(jnp.maximum(pre[1], 0.0), w_ref[DH:, :],
                   preferred_element_type=jnp.float32)) * dinv_ref[...]
    _write_halves(o_ref, res)


_t2_call = pl.pallas_call(
    _t2_body,
    grid=(GRID,),
    in_specs=[
        pl.BlockSpec((R, D), lambda i: (i, 0)),
        pl.BlockSpec((NC, R, D), lambda i: (0, i, 0)),
        pl.BlockSpec((2, R, DH), lambda i: (0, i, 0)),
        pl.BlockSpec((1, D), lambda i: (0, 0)),
        pl.BlockSpec((D, D), lambda i: (0, 0)),
    ],
    out_specs=pl.BlockSpec((2, R, DH), lambda i: (0, i, 0)),
    out_shape=jax.ShapeDtypeStruct((2, N, DH), jnp.float32),
)


def _t3_body(dinv_ref, a_ref, h_ref, b_ref, x_ref, g_ref, be_ref, o_ref):
    pre = _pre_halves(dinv_ref, a_ref, h_ref, b_ref)
    y = [x_ref[:, k * DH:(k + 1) * DH] + pre[k] for k in range(2)]
    mu = (jnp.sum(y[0], axis=-1, keepdims=True) +
          jnp.sum(y[1], axis=-1, keepdims=True)) * (1.0 / D)
    d = [y[k] - mu for k in range(2)]
    var = (jnp.sum(d[0] * d[0], axis=-1, keepdims=True) +
           jnp.sum(d[1] * d[1], axis=-1, keepdims=True)) * (1.0 / D)
    rstd = lax.rsqrt(var + 1e-5)
    for k in range(2):
        o_ref[:, k * DH:(k + 1) * DH] = (
            d[k] * rstd * g_ref[0, k * DH:(k + 1) * DH][None, :]
            + be_ref[0, k * DH:(k + 1) * DH][None, :])


_t3_call = pl.pallas_call(
    _t3_body,
    grid=(GRID,),
    in_specs=[
        pl.BlockSpec((R, D), lambda i: (i, 0)),
        pl.BlockSpec((NC, R, D), lambda i: (0, i, 0)),
        pl.BlockSpec((2, R, DH), lambda i: (0, i, 0)),
        pl.BlockSpec((1, D), lambda i: (0, 0)),
        pl.BlockSpec((R, D), lambda i: (i, 0)),
        pl.BlockSpec((1, D), lambda i: (0, 0)),
        pl.BlockSpec((1, D), lambda i: (0, 0)),
    ],
    out_specs=pl.BlockSpec((R, D), lambda i: (i, 0)),
    out_shape=jax.ShapeDtypeStruct((N, D), jnp.float32),
)


@jax.jit
def kernel(x, edge_index, W1, b1, W2, b2, gamma, beta):
    pad = E_P - E
    src_p = jnp.concatenate([edge_index[0],
                             jnp.zeros((pad,), edge_index.dtype)])
    dst_p = jnp.concatenate([edge_index[1],
                             jnp.full((pad,), N, edge_index.dtype)])
    src_m = src_p.reshape(NC, NS, NCH, C)
    dst_m = dst_p.reshape(NC, NS, NCH, C)

    deg_parts = _deg_call()(dst_m).reshape(NC, N, 16)
    h1, dinv_b = _t1_call(deg_parts, x, W1)
    acc1 = _layer_call()(h1, src_m, dst_m)
    h2 = _t2_call(dinv_b, acc1, h1, b1.reshape(1, D), W2)
    acc2 = _layer_call()(h2, src_m, dst_m)
    return _t3_call(dinv_b, acc2, h2, b2.reshape(1, D), x,
                    gamma.reshape(1, D), beta.reshape(1, D))


# speedup vs baseline: 3.2008x; 3.2008x over previous
import functools

import jax
import jax.numpy as jnp
from jax import lax
from jax.experimental import pallas as pl
from jax.experimental.pallas import tpu as pltpu
from jax.experimental.pallas import tpu_sc as plsc

N = 10000
E = 320000
D = 128

NC = 2
NS = 16
NW = NC * NS
EPT = E // NW
RPT = N // NS

C = 125
NCH = EPT // C
NBUF = 8
NROW = N
RPT_Z = RPT

def _deg_body(dst_hbm, out_hbm, dst_v, ones_v, zero_v, acc_sh):
    c = lax.axis_index("c")
    s = lax.axis_index("s")
    pltpu.sync_copy(dst_hbm.at[c, s], dst_v)

    def fill_ones(i, _):
        ones_v[i, :] = jnp.full((16,), 1.0, jnp.float32)
        return 0

    lax.fori_loop(0, C, fill_ones, 0)

    def fill_zero(i, _):
        zero_v[i, :] = jnp.zeros((16,), jnp.float32)
        return 0

    lax.fori_loop(0, 25, fill_zero, 0)

    def zcp(k, _):
        pltpu.sync_copy(zero_v, acc_sh.at[pl.ds(s * RPT_Z + k * 25, 25)])
        return 0

    lax.fori_loop(0, RPT_Z // 25, zcp, 0)
    plsc.subcore_barrier()

    def body(j, _):
        pltpu.sync_copy(ones_v, acc_sh.at[dst_v.at[j]], add=True)
        return 0

    lax.fori_loop(0, NCH, body, 0)
    plsc.subcore_barrier()
    pltpu.sync_copy(acc_sh.at[pl.ds(s * RPT, RPT)], out_hbm.at[c, s])


@functools.cache
def _deg_call():
    return pl.kernel(
        _deg_body,
        out_type=jax.ShapeDtypeStruct((NC, NS, RPT, 16), jnp.float32),
        mesh=plsc.VectorSubcoreMesh(core_axis_name="c", subcore_axis_name="s"),
        scratch_types=[
            pltpu.VMEM((NCH, C), jnp.int32),
            pltpu.VMEM((C, 16), jnp.float32),
            pltpu.VMEM((25, 16), jnp.float32),
            pltpu.VMEM_SHARED((NROW, 16), jnp.float32),
        ],
        compiler_params=pltpu.CompilerParams(use_tc_tiling_on_sc=False),
    )


DH = D // 2


def _layer_body(h_hbm, src_hbm, dst_hbm, out_hbm, src_v, dst_v, rows_v, zero_v,
                acc_sh, sem0, sem1):
    c = lax.axis_index("c")
    s = lax.axis_index("s")
    pltpu.sync_copy(src_hbm.at[c, s], src_v)
    pltpu.sync_copy(dst_hbm.at[c, s], dst_v)

    def fz(i, _):
        for jj in range(DH // 16):
            zero_v[i, pl.ds(jj * 16, 16)] = jnp.zeros((16,), jnp.float32)
        return 0

    lax.fori_loop(0, 25, fz, 0)

    for half in range(2):
        def zcp(k, _):
            pltpu.sync_copy(zero_v, acc_sh.at[pl.ds(s * RPT_Z + k * 25, 25)])
            return 0

        lax.fori_loop(0, RPT_Z // 25, zcp, 0)
        plsc.subcore_barrier()

        h_half = h_hbm.at[half]
        for b in range(NBUF):
            pltpu.async_copy(h_half.at[src_v.at[b]], rows_v.at[b], sem0)

        def body(g, _):
            j0 = NBUF * g
            for b in range(NBUF):
                pltpu.make_async_copy(h_half.at[src_v.at[j0 + b]],
                                      rows_v.at[b], sem0).wait()
                pltpu.async_copy(rows_v.at[b], acc_sh.at[dst_v.at[j0 + b]],
                                 sem1, add=True)
            for b in range(NBUF):
                nj = j0 + NBUF + b

                @pl.when(nj < NCH)
                def _():
                    pltpu.make_async_copy(rows_v.at[b],
                                          acc_sh.at[dst_v.at[j0 + b]],
                                          sem1).wait()
                    pltpu.async_copy(h_half.at[src_v.at[nj]], rows_v.at[b], sem0)

            return 0

        lax.fori_loop(0, NCH // NBUF, body, 0)
        for b in range(NBUF):
            pltpu.make_async_copy(rows_v.at[b], acc_sh.at[dst_v.at[b]],
                                  sem1).wait()
        plsc.subcore_barrier()
        pltpu.sync_copy(
            acc_sh.at[pl.ds(s * RPT, RPT)],
            out_hbm.at[c, pl.ds(s * RPT, RPT), pl.ds(half * DH, DH)])


@functools.cache
def _layer_call():
    return pl.kernel(
        _layer_body,
        out_type=jax.ShapeDtypeStruct((NC, N, D), jnp.float32),
        mesh=plsc.VectorSubcoreMesh(core_axis_name="c", subcore_axis_name="s"),
        scratch_types=[
            pltpu.VMEM((NCH, C), jnp.int32),
            pltpu.VMEM((NCH, C), jnp.int32),
            pltpu.VMEM((NBUF, C, DH), jnp.float32),
            pltpu.VMEM((25, DH), jnp.float32),
            pltpu.VMEM_SHARED((NROW, DH), jnp.float32),
            pltpu.SemaphoreType.DMA,
            pltpu.SemaphoreType.DMA,
        ],
        compiler_params=pltpu.CompilerParams(use_tc_tiling_on_sc=False),
    )


R = 1000
GRID = N // R


def _dinv_of(dp_ref):
    deg = dp_ref[0, :, 0:1] + dp_ref[1, :, 0:1] + 1.0
    return lax.rsqrt(deg)


def _write_halves(o_ref, res):
    o_ref[0] = res[:, :DH]
    o_ref[1] = res[:, DH:]


def _pre_halves(dinv_ref, a_ref, h_ref, b_ref):
    asum = a_ref[0] + a_ref[1]
    return [
        (asum[:, k * DH:(k + 1) * DH] + h_ref[k])
        * dinv_ref[:, k * DH:(k + 1) * DH]
        + b_ref[0, k * DH:(k + 1) * DH][None, :]
        for k in range(2)
    ]


def _t1_body(dp_ref, x_ref, w_ref, o_ref, dinv_ref):
    dinv = _dinv_of(dp_ref)
    res = jnp.dot(x_ref[...], w_ref[...],
                  preferred_element_type=jnp.float32) * dinv
    _write_halves(o_ref, res)
    dinv_ref[...] = jnp.broadcast_to(dinv, (R, D))


_t1_call = pl.pallas_call(
    _t1_body,
    grid=(GRID,),
    in_specs=[
        pl.BlockSpec((NC, R, 16), lambda i: (0, i, 0)),
        pl.BlockSpec((R, D), lambda i: (i, 0)),
        pl.BlockSpec((D, D), lambda i: (0, 0)),
    ],
    out_specs=[
        pl.BlockSpec((2, R, DH), lambda i: (0, i, 0)),
        pl.BlockSpec((R, D), lambda i: (i, 0)),
    ],
    out_shape=[
        jax.ShapeDtypeStruct((2, N, DH), jnp.float32),
        jax.ShapeDtypeStruct((N, D), jnp.float32),
    ],
)


def _t2_body(dinv_ref, a_ref, h_ref, b_ref, w_ref, o_ref):
    pre = _pre_halves(dinv_ref, a_ref, h_ref, b_ref)
    res = (jnp.dot(jnp.maximum(pre[0], 0.0), w_ref[:DH, :],
                   preferred_element_type=jnp.float32) +
           jnp.dot(jnp.maximum(pre[1], 0.0), w_ref[DH:, :],
                   preferred_element_type=jnp.float32)) * dinv_ref[...]
    _write_halves(o_ref, res)


_t2_call = pl.pallas_call(
    _t2_body,
    grid=(GRID,),
    in_specs=[
        pl.BlockSpec((R, D), lambda i: (i, 0)),
        pl.BlockSpec((NC, R, D), lambda i: (0, i, 0)),
        pl.BlockSpec((2, R, DH), lambda i: (0, i, 0)),
        pl.BlockSpec((1, D), lambda i: (0, 0)),
        pl.BlockSpec((D, D), lambda i: (0, 0)),
    ],
    out_specs=pl.BlockSpec((2, R, DH), lambda i: (0, i, 0)),
    out_shape=jax.ShapeDtypeStruct((2, N, DH), jnp.float32),
)


def _t3_body(dinv_ref, a_ref, h_ref, b_ref, x_ref, g_ref, be_ref, o_ref):
    pre = _pre_halves(dinv_ref, a_ref, h_ref, b_ref)
    y = [x_ref[:, k * DH:(k + 1) * DH] + pre[k] for k in range(2)]
    mu = (jnp.sum(y[0], axis=-1, keepdims=True) +
          jnp.sum(y[1], axis=-1, keepdims=True)) * (1.0 / D)
    d = [y[k] - mu for k in range(2)]
    var = (jnp.sum(d[0] * d[0], axis=-1, keepdims=True) +
           jnp.sum(d[1] * d[1], axis=-1, keepdims=True)) * (1.0 / D)
    rstd = lax.rsqrt(var + 1e-5)
    for k in range(2):
        o_ref[:, k * DH:(k + 1) * DH] = (
            d[k] * rstd * g_ref[0, k * DH:(k + 1) * DH][None, :]
            + be_ref[0, k * DH:(k + 1) * DH][None, :])


_t3_call = pl.pallas_call(
    _t3_body,
    grid=(GRID,),
    in_specs=[
        pl.BlockSpec((R, D), lambda i: (i, 0)),
        pl.BlockSpec((NC, R, D), lambda i: (0, i, 0)),
        pl.BlockSpec((2, R, DH), lambda i: (0, i, 0)),
        pl.BlockSpec((1, D), lambda i: (0, 0)),
        pl.BlockSpec((R, D), lambda i: (i, 0)),
        pl.BlockSpec((1, D), lambda i: (0, 0)),
        pl.BlockSpec((1, D), lambda i: (0, 0)),
    ],
    out_specs=pl.BlockSpec((R, D), lambda i: (i, 0)),
    out_shape=jax.ShapeDtypeStruct((N, D), jnp.float32),
)


@jax.jit
def kernel(x, edge_index, W1, b1, W2, b2, gamma, beta):
    src_m = edge_index[0].reshape(NC, NS, NCH, C)
    dst_m = edge_index[1].reshape(NC, NS, NCH, C)

    deg_parts = _deg_call()(dst_m).reshape(NC, N, 16)
    h1, dinv_b = _t1_call(deg_parts, x, W1)
    acc1 = _layer_call()(h1, src_m, dst_m)
    h2 = _t2_call(dinv_b, acc1, h1, b1.reshape(1, D), W2)
    acc2 = _layer_call()(h2, src_m, dst_m)
    return _t3_call(dinv_b, acc2, h2, b2.reshape(1, D), x,
                    gamma.reshape(1, D), beta.reshape(1, D))


# speedup vs baseline: 3.2414x; 1.0127x over previous
import functools

import jax
import jax.numpy as jnp
from jax import lax
from jax.experimental import pallas as pl
from jax.experimental.pallas import tpu as pltpu
from jax.experimental.pallas import tpu_sc as plsc

N = 10000
E = 320000
D = 128

NC = 2
NS = 16
NW = NC * NS
EPT = E // NW
RPT = N // NS

C = 125
NCH = EPT // C
NBUF = 8
NROW = N
RPT_Z = RPT

def _deg_body(dst_hbm, out_hbm, dst_v, ones_v, zero_v, acc_sh, sem):
    c = lax.axis_index("c")
    s = lax.axis_index("s")
    pltpu.sync_copy(dst_hbm.at[c, s], dst_v)

    def fill_ones(i, _):
        ones_v[i, :] = jnp.full((16,), 1.0, jnp.float32)
        return 0

    lax.fori_loop(0, C, fill_ones, 0)

    def fill_zero(i, _):
        zero_v[i, :] = jnp.zeros((16,), jnp.float32)
        return 0

    lax.fori_loop(0, 25, fill_zero, 0)

    def zcp(k, _):
        pltpu.sync_copy(zero_v, acc_sh.at[pl.ds(s * RPT_Z + k * 25, 25)])
        return 0

    lax.fori_loop(0, RPT_Z // 25, zcp, 0)
    plsc.subcore_barrier()

    for b in range(NBUF):
        pltpu.async_copy(ones_v, acc_sh.at[dst_v.at[b]], sem, add=True)

    def body(j, _):
        pltpu.make_async_copy(ones_v, acc_sh.at[dst_v.at[j]], sem).wait()

        @pl.when(j + NBUF < NCH)
        def _():
            pltpu.async_copy(ones_v, acc_sh.at[dst_v.at[j + NBUF]], sem,
                             add=True)

        return 0

    lax.fori_loop(0, NCH, body, 0)
    plsc.subcore_barrier()
    pltpu.sync_copy(acc_sh.at[pl.ds(s * RPT, RPT)], out_hbm.at[c, s])


@functools.cache
def _deg_call():
    return pl.kernel(
        _deg_body,
        out_type=jax.ShapeDtypeStruct((NC, NS, RPT, 16), jnp.float32),
        mesh=plsc.VectorSubcoreMesh(core_axis_name="c", subcore_axis_name="s"),
        scratch_types=[
            pltpu.VMEM((NCH, C), jnp.int32),
            pltpu.VMEM((C, 16), jnp.float32),
            pltpu.VMEM((25, 16), jnp.float32),
            pltpu.VMEM_SHARED((NROW, 16), jnp.float32),
            pltpu.SemaphoreType.DMA,
        ],
        compiler_params=pltpu.CompilerParams(use_tc_tiling_on_sc=False),
    )


DH = D // 2


def _layer_body(h_hbm, src_hbm, dst_hbm, out_hbm, src_v, dst_v, rows_v, zero_v,
                acc_sh, sem0, sem1):
    c = lax.axis_index("c")
    s = lax.axis_index("s")
    pltpu.sync_copy(src_hbm.at[c, s], src_v)
    pltpu.sync_copy(dst_hbm.at[c, s], dst_v)

    def fz(i, _):
        for jj in range(DH // 16):
            zero_v[i, pl.ds(jj * 16, 16)] = jnp.zeros((16,), jnp.float32)
        return 0

    lax.fori_loop(0, 25, fz, 0)

    for half in range(2):
        def zcp(k, _):
            pltpu.sync_copy(zero_v, acc_sh.at[pl.ds(s * RPT_Z + k * 25, 25)])
            return 0

        lax.fori_loop(0, RPT_Z // 25, zcp, 0)
        plsc.subcore_barrier()

        h_half = h_hbm.at[half]
        for b in range(NBUF):
            pltpu.async_copy(h_half.at[src_v.at[b]], rows_v.at[b], sem0)

        def body(g, _):
            j0 = NBUF * g
            for b in range(NBUF):
                pltpu.make_async_copy(h_half.at[src_v.at[j0 + b]],
                                      rows_v.at[b], sem0).wait()
                pltpu.async_copy(rows_v.at[b], acc_sh.at[dst_v.at[j0 + b]],
                                 sem1, add=True)
            for b in range(NBUF):
                nj = j0 + NBUF + b

                @pl.when(nj < NCH)
                def _():
                    pltpu.make_async_copy(rows_v.at[b],
                                          acc_sh.at[dst_v.at[j0 + b]],
                                          sem1).wait()
                    pltpu.async_copy(h_half.at[src_v.at[nj]], rows_v.at[b], sem0)

            return 0

        lax.fori_loop(0, NCH // NBUF, body, 0)
        for b in range(NBUF):
            pltpu.make_async_copy(rows_v.at[b], acc_sh.at[dst_v.at[b]],
                                  sem1).wait()
        plsc.subcore_barrier()
        pltpu.sync_copy(
            acc_sh.at[pl.ds(s * RPT, RPT)],
            out_hbm.at[c, pl.ds(s * RPT, RPT), pl.ds(half * DH, DH)])


@functools.cache
def _layer_call():
    return pl.kernel(
        _layer_body,
        out_type=jax.ShapeDtypeStruct((NC, N, D), jnp.float32),
        mesh=plsc.VectorSubcoreMesh(core_axis_name="c", subcore_axis_name="s"),
        scratch_types=[
            pltpu.VMEM((NCH, C), jnp.int32),
            pltpu.VMEM((NCH, C), jnp.int32),
            pltpu.VMEM((NBUF, C, DH), jnp.float32),
            pltpu.VMEM((25, DH), jnp.float32),
            pltpu.VMEM_SHARED((NROW, DH), jnp.float32),
            pltpu.SemaphoreType.DMA,
            pltpu.SemaphoreType.DMA,
        ],
        compiler_params=pltpu.CompilerParams(use_tc_tiling_on_sc=False),
    )


R = 1000
GRID = N // R


def _dinv_of(dp_ref):
    deg = dp_ref[0, :, 0:1] + dp_ref[1, :, 0:1] + 1.0
    return lax.rsqrt(deg)


def _write_halves(o_ref, res):
    o_ref[0] = res[:, :DH]
    o_ref[1] = res[:, DH:]


def _pre_halves(dinv_ref, a_ref, h_ref, b_ref):
    asum = a_ref[0] + a_ref[1]
    return [
        (asum[:, k * DH:(k + 1) * DH] + h_ref[k])
        * dinv_ref[:, k * DH:(k + 1) * DH]
        + b_ref[0, k * DH:(k + 1) * DH][None, :]
        for k in range(2)
    ]


def _t1_body(dp_ref, x_ref, w_ref, o_ref, dinv_ref):
    dinv = _dinv_of(dp_ref)
    res = jnp.dot(x_ref[...], w_ref[...],
                  preferred_element_type=jnp.float32) * dinv
    _write_halves(o_ref, res)
    dinv_ref[...] = jnp.broadcast_to(dinv, (R, D))


_t1_call = pl.pallas_call(
    _t1_body,
    grid=(GRID,),
    in_specs=[
        pl.BlockSpec((NC, R, 16), lambda i: (0, i, 0)),
        pl.BlockSpec((R, D), lambda i: (i, 0)),
        pl.BlockSpec((D, D), lambda i: (0, 0)),
    ],
    out_specs=[
        pl.BlockSpec((2, R, DH), lambda i: (0, i, 0)),
        pl.BlockSpec((R, D), lambda i: (i, 0)),
    ],
    out_shape=[
        jax.ShapeDtypeStruct((2, N, DH), jnp.float32),
        jax.ShapeDtypeStruct((N, D), jnp.float32),
    ],
)


def _t2_body(dinv_ref, a_ref, h_ref, b_ref, w_ref, o_ref):
    pre = _pre_halves(dinv_ref, a_ref, h_ref, b_ref)
    res = (jnp.dot(jnp.maximum(pre[0], 0.0), w_ref[:DH, :],
                   preferred_element_type=jnp.float32) +
           jnp.dot(jnp.maximum(pre[1], 0.0), w_ref[DH:, :],
                   preferred_element_type=jnp.float32)) * dinv_ref[...]
    _write_halves(o_ref, res)


_t2_call = pl.pallas_call(
    _t2_body,
    grid=(GRID,),
    in_specs=[
        pl.BlockSpec((R, D), lambda i: (i, 0)),
        pl.BlockSpec((NC, R, D), lambda i: (0, i, 0)),
        pl.BlockSpec((2, R, DH), lambda i: (0, i, 0)),
        pl.BlockSpec((1, D), lambda i: (0, 0)),
        pl.BlockSpec((D, D), lambda i: (0, 0)),
    ],
    out_specs=pl.BlockSpec((2, R, DH), lambda i: (0, i, 0)),
    out_shape=jax.ShapeDtypeStruct((2, N, DH), jnp.float32),
)


def _t3_body(dinv_ref, a_ref, h_ref, b_ref, x_ref, g_ref, be_ref, o_ref):
    pre = _pre_halves(dinv_ref, a_ref, h_ref, b_ref)
    y = [x_ref[:, k * DH:(k + 1) * DH] + pre[k] for k in range(2)]
    mu = (jnp.sum(y[0], axis=-1, keepdims=True) +
          jnp.sum(y[1], axis=-1, keepdims=True)) * (1.0 / D)
    d = [y[k] - mu for k in range(2)]
    var = (jnp.sum(d[0] * d[0], axis=-1, keepdims=True) +
           jnp.sum(d[1] * d[1], axis=-1, keepdims=True)) * (1.0 / D)
    rstd = lax.rsqrt(var + 1e-5)
    for k in range(2):
        o_ref[:, k * DH:(k + 1) * DH] = (
            d[k] * rstd * g_ref[0, k * DH:(k + 1) * DH][None, :]
            + be_ref[0, k * DH:(k + 1) * DH][None, :])


_t3_call = pl.pallas_call(
    _t3_body,
    grid=(GRID,),
    in_specs=[
        pl.BlockSpec((R, D), lambda i: (i, 0)),
        pl.BlockSpec((NC, R, D), lambda i: (0, i, 0)),
        pl.BlockSpec((2, R, DH), lambda i: (0, i, 0)),
        pl.BlockSpec((1, D), lambda i: (0, 0)),
        pl.BlockSpec((R, D), lambda i: (i, 0)),
        pl.BlockSpec((1, D), lambda i: (0, 0)),
        pl.BlockSpec((1, D), lambda i: (0, 0)),
    ],
    out_specs=pl.BlockSpec((R, D), lambda i: (i, 0)),
    out_shape=jax.ShapeDtypeStruct((N, D), jnp.float32),
)


@jax.jit
def kernel(x, edge_index, W1, b1, W2, b2, gamma, beta):
    src_m = edge_index[0].reshape(NC, NS, NCH, C)
    dst_m = edge_index[1].reshape(NC, NS, NCH, C)

    deg_parts = _deg_call()(dst_m).reshape(NC, N, 16)
    h1, dinv_b = _t1_call(deg_parts, x, W1)
    acc1 = _layer_call()(h1, src_m, dst_m)
    h2 = _t2_call(dinv_b, acc1, h1, b1.reshape(1, D), W2)
    acc2 = _layer_call()(h2, src_m, dst_m)
    return _t3_call(dinv_b, acc2, h2, b2.reshape(1, D), x,
                    gamma.reshape(1, D), beta.reshape(1, D))
